# Initial kernel scaffold; baseline (speedup 1.0000x reference)
import jax
import jax.numpy as jnp
from jax.experimental import pallas as pl


def kernel(x):
    raise NotImplementedError("write your pallas kernel here")


# speedup vs baseline: 1.6247x; 1.6247x over previous
import functools

import jax
import jax.numpy as jnp
from jax import lax
from jax.experimental import pallas as pl
from jax.experimental.pallas import tpu as pltpu
from jax.experimental.pallas import tpu_sc as plsc

R = 64
N = 8192

_info = plsc.get_sparse_core_info()
_NC, _NS, _L = _info.num_cores, _info.num_subcores, _info.num_lanes
_NW = _NC * _NS
_CPW = N // _NW
_G = _CPW // _L


@functools.partial(
    pl.kernel,
    mesh=plsc.VectorSubcoreMesh(core_axis_name="c", subcore_axis_name="s"),
    out_type=(
        jax.ShapeDtypeStruct((N,), jnp.float32),
        jax.ShapeDtypeStruct((N,), jnp.int32),
    ),
    scratch_types=[
        pltpu.VMEM((R, _CPW), jnp.float32),
        pltpu.VMEM((_CPW,), jnp.float32),
        pltpu.VMEM((_CPW,), jnp.int32),
    ],
)
def _colmax(x_hbm, vals_hbm, idx_hbm, x_v, mv_v, mi_v):
    wid = lax.axis_index("s") * _NC + lax.axis_index("c")
    base = wid * _CPW
    pltpu.sync_copy(x_hbm.at[:, pl.ds(base, _CPW)], x_v)

    def group(g, carry):
        cols = pl.ds(g * _L, _L)
        m = x_v[0, cols]
        idx = jnp.zeros((_L,), jnp.int32)
        for r in range(1, R):
            v = x_v[r, cols]
            pred = v > m
            m = jnp.where(pred, v, m)
            idx = jnp.where(pred, jnp.full((_L,), r, jnp.int32), idx)
        mv_v[cols] = m
        mi_v[cols] = idx
        return carry

    lax.fori_loop(0, _G, group, 0)

    pltpu.sync_copy(mv_v, vals_hbm.at[pl.ds(base, _CPW)])
    pltpu.sync_copy(mi_v, idx_hbm.at[pl.ds(base, _CPW)])


def kernel(x):
    vals, idx = _colmax(x)
    return vals.reshape(1, N), idx.reshape(1, N).astype(jnp.int64)
